# baseline (speedup 1.0000x reference)
import functools

import jax
import jax.numpy as jnp
from jax import lax
from jax.experimental import pallas as pl
from jax.experimental.pallas import tpu as pltpu
from jax.experimental.pallas import tpu_sc as plsc

B = 16
N = 100000
D = 32
J = 200
CPW = 8
NWORK = J // CPW

_mesh = plsc.VectorSubcoreMesh(core_axis_name="c", subcore_axis_name="s")


@functools.partial(
    pl.kernel,
    mesh=_mesh,
    compiler_params=pltpu.CompilerParams(
        needs_layout_passes=False, use_tc_tiling_on_sc=False),
    out_type=jax.ShapeDtypeStruct((B * J, D), jnp.float32),
    scratch_types=[
        pltpu.VMEM((B * J,), jnp.int32),
        pltpu.VMEM((CPW * B,), jnp.int32),
        pltpu.VMEM((CPW * B,), jnp.int32),
        pltpu.VMEM((CPW * B, D), jnp.float32),
        pltpu.SemaphoreType.DMA,
        pltpu.SemaphoreType.DMA,
    ],
)
def _select_kernel(x_hbm, idx_hbm, out_hbm, idx_v, gidx_v, didx_v, rows_v,
                   gsem, ssem):
    wid = lax.axis_index("s") * 2 + lax.axis_index("c")

    @pl.when(wid < NWORK)
    def _():
        pltpu.sync_copy(idx_hbm, idx_v)
        lanes = lax.iota(jnp.int32, 16)
        j0 = wid * CPW
        for c in range(CPW):
            j = j0 + c
            col = plsc.load_gather(idx_v, [lanes * J + j])
            srt = lax.sort(col)
            gidx_v[pl.ds(c * B, B)] = srt + lanes * N
            didx_v[pl.ds(c * B, B)] = lanes * J + j
        pltpu.async_copy(x_hbm.at[gidx_v], rows_v, gsem).wait()
        pltpu.async_copy(rows_v, out_hbm.at[didx_v], ssem).wait()


@jax.jit
def kernel(X, idx):
    Xf = X.reshape(B * N, D)
    idxf = idx.astype(jnp.int32).reshape(-1)
    out = _select_kernel(Xf, idxf)
    return out.reshape(B, J, D)


# speedup vs baseline: 15.5685x; 15.5685x over previous
import functools

import jax
import jax.numpy as jnp
from jax import lax
from jax.experimental import pallas as pl
from jax.experimental.pallas import tpu as pltpu
from jax.experimental.pallas import tpu_sc as plsc

B = 16
N = 100000
D = 32
J = 200
CPG = 8
NWORK = J // CPG
RING = 16

_mesh = plsc.VectorSubcoreMesh(core_axis_name="c", subcore_axis_name="s")


@functools.partial(
    pl.kernel,
    mesh=_mesh,
    compiler_params=pltpu.CompilerParams(
        needs_layout_passes=False, use_tc_tiling_on_sc=True),
    out_type=jax.ShapeDtypeStruct((NWORK, D, 128), jnp.float32),
    scratch_types=[
        pltpu.VMEM((B * J,), jnp.int32),
        pltpu.VMEM((RING, D, 128), jnp.float32),
        pltpu.VMEM((D, 128), jnp.float32),
        pltpu.SemaphoreType.DMA,
        pltpu.SemaphoreType.DMA,
    ],
)
def _select_kernel(x_hbm, idx_hbm, out_hbm, idx_v, slab_v, out_v, gsem, osem):
    wid = lax.axis_index("s") * 2 + lax.axis_index("c")

    @pl.when(wid < NWORK)
    def _():
        pltpu.sync_copy(idx_hbm, idx_v)
        lanes = lax.iota(jnp.int32, 16)
        j0 = wid * CPG
        ns = []
        for c in range(CPG):
            col = plsc.load_gather(idx_v, [lanes * J + j0 + c])
            srt = lax.sort(col)
            for b in range(B):
                ns.append(srt[b])

        def issue(g):
            n = ns[g]
            t = pl.multiple_of(n & -128, 128)
            b = g % B
            return pltpu.async_copy(
                x_hbm.at[b, :, pl.ds(t, 128)], slab_v.at[g % RING], gsem)

        def extract(g, cp):
            cp.wait()
            n = ns[g]
            off = jnp.broadcast_to(n & 127, (16,)).astype(jnp.int32)
            gv = jnp.full((16,), g % RING, jnp.int32)
            v0 = plsc.load_gather(slab_v, [gv, lanes, off])
            v1 = plsc.load_gather(slab_v, [gv, lanes + 16, off])
            c, b = g // B, g % B
            base = c * 512 + b * 32
            r, q = base // 128, base % 128
            out_v[r, pl.ds(q, 16)] = v0
            out_v[r, pl.ds(q + 16, 16)] = v1

        total = CPG * B
        pending = [issue(g) for g in range(RING)]
        for g in range(total):
            extract(g, pending[g % RING])
            if g + RING < total:
                pending[(g + RING) % RING] = issue(g + RING)
        pltpu.async_copy(out_v, out_hbm.at[wid], osem).wait()


@jax.jit
def kernel(X, idx):
    Xt = X.transpose(0, 2, 1)
    idxf = idx.astype(jnp.int32).reshape(-1)
    out25 = _select_kernel(Xt, idxf)
    out = out25.reshape(NWORK, CPG, B, D).transpose(2, 0, 1, 3)
    return out.reshape(B, J, D)
